# baseline (speedup 1.0000x reference)
import functools

import jax
import jax.numpy as jnp
from jax import lax
from jax.experimental import pallas as pl
from jax.experimental.pallas import tpu as pltpu
from jax.experimental.pallas import tpu_sc as plsc

_L = 16


def _fast_rsqrt(x):
    i = lax.bitcast_convert_type(x, jnp.int32)
    i = jnp.int32(0x5F3759DF) - lax.shift_right_arithmetic(i, 1)
    y = lax.bitcast_convert_type(i, jnp.float32)
    for _ in range(3):
        y = y * (1.5 - 0.5 * x * y * y)
    return y


def _make_sc_kernel(B, T, C, K):
    assert C % _L == 0
    n_groups = T // _L
    c_chunks = C // _L
    mesh = plsc.VectorSubcoreMesh(
        core_axis_name="c", subcore_axis_name="s", num_cores=2, num_subcores=16
    )

    @functools.partial(
        pl.kernel,
        out_type=jax.ShapeDtypeStruct((B, C), jnp.float32),
        mesh=mesh,
        scratch_types=[
            pltpu.VMEM((T * C,), jnp.float32),
            pltpu.VMEM((T * K,), jnp.int32),
            pltpu.VMEM((T,), jnp.float32),
            pltpu.VMEM((C,), jnp.float32),
        ],
        compiler_params=pltpu.CompilerParams(needs_layout_passes=False),
    )
    def sc_kernel(x_hbm, idx_hbm, out_hbm, x_v, idx_v, w_v, o_v):
        cid = lax.axis_index("c")
        sid = lax.axis_index("s")
        b = sid * 2 + cid

        pltpu.sync_copy(x_hbm.at[b], x_v)
        pltpu.sync_copy(idx_hbm, idx_v)

        zero16 = jnp.zeros((_L,), jnp.float32)

        def zero_body(i, carry):
            w_v[pl.ds(i * _L, _L)] = zero16
            return carry

        lax.fori_loop(0, T // _L, zero_body, 0)

        lane = lax.iota(jnp.int32, _L)

        def group_body(g, carry):
            t0 = g * _L
            tvec = t0 + lane
            qbase = tvec * C
            tk = tvec * K
            idxv = [plsc.load_gather(idx_v, [tk + j]) for j in range(K)]
            idxw = [iv * C for iv in idxv]

            def c_body(c, acc):
                q2 = acc[0]
                dots = list(acc[1:1 + K])
                n2s = list(acc[1 + K:])
                qv = plsc.load_gather(x_v, [qbase + c])
                q2 = q2 + qv * qv
                for j in range(K):
                    nv = plsc.load_gather(x_v, [idxw[j] + c])
                    dots[j] = dots[j] + qv * nv
                    n2s[j] = n2s[j] + nv * nv
                return tuple([q2] + dots + n2s)

            init = tuple(zero16 for _ in range(1 + 2 * K))
            acc = lax.fori_loop(0, C, c_body, init, unroll=4)
            q2 = acc[0]
            dots = acc[1:1 + K]
            n2s = acc[1 + K:]

            sims = [
                dots[j] * _fast_rsqrt(jnp.maximum(q2 * n2s[j], 1e-16))
                for j in range(K)
            ]
            m = sims[0]
            for j in range(1, K):
                m = jnp.maximum(m, sims[j])
            exps = [jnp.exp(s - m) for s in sims]
            tot = exps[0]
            for j in range(1, K):
                tot = tot + exps[j]
            r = 1.0 / tot
            for j in range(K):
                plsc.addupdate_scatter(w_v, [idxv[j]], exps[j] * r)
            return carry

        lax.fori_loop(0, n_groups, group_body, 0)

        def row_body(s, acc):
            wv = plsc.load_gather(w_v, [jnp.full((_L,), s, jnp.int32)])
            base = s * C
            return tuple(
                acc[k] + wv * x_v[pl.ds(base + k * _L, _L)]
                for k in range(c_chunks)
            )

        acc = lax.fori_loop(
            0, T, row_body, tuple(zero16 for _ in range(c_chunks)), unroll=4
        )
        scale = jnp.float32(1.0 / T)
        for k in range(c_chunks):
            o_v[pl.ds(k * _L, _L)] = acc[k] * scale
        pltpu.sync_copy(o_v, out_hbm.at[b])

    return sc_kernel


def kernel(X, neighbor_idx):
    B, T, C = X.shape
    K = neighbor_idx.shape[1]
    x_flat = X.reshape(B, T * C)
    idx_flat = neighbor_idx.astype(jnp.int32).reshape(T * K)
    out = _make_sc_kernel(B, T, C, K)(x_flat, idx_flat)
    return out.reshape(B, 1, C)


# speedup vs baseline: 2.3283x; 2.3283x over previous
import functools

import jax
import jax.numpy as jnp
from jax import lax
from jax.experimental import pallas as pl
from jax.experimental.pallas import tpu as pltpu
from jax.experimental.pallas import tpu_sc as plsc

_L = 16


def _fast_rsqrt(x):
    i = lax.bitcast_convert_type(x, jnp.int32)
    i = jnp.int32(0x5F3759DF) - lax.shift_right_arithmetic(i, 1)
    y = lax.bitcast_convert_type(i, jnp.float32)
    for _ in range(3):
        y = y * (1.5 - 0.5 * x * y * y)
    return y


def _make_sc_kernel(B, T, C, K):
    assert C % _L == 0
    n_groups = T // _L
    c_chunks = C // _L
    mesh = plsc.VectorSubcoreMesh(
        core_axis_name="c", subcore_axis_name="s", num_cores=2, num_subcores=16
    )

    @functools.partial(
        pl.kernel,
        out_type=jax.ShapeDtypeStruct((B, C), jnp.float32),
        mesh=mesh,
        scratch_types=[
            pltpu.VMEM((T, C + 1), jnp.float32),
            pltpu.VMEM((T * K,), jnp.int32),
            pltpu.VMEM((T,), jnp.float32),
            pltpu.VMEM((C,), jnp.float32),
        ],
        compiler_params=pltpu.CompilerParams(
            needs_layout_passes=False, use_tc_tiling_on_sc=False
        ),
    )
    def sc_kernel(x_hbm, idx_hbm, out_hbm, x_v, idx_v, w_v, o_v):
        cid = lax.axis_index("c")
        sid = lax.axis_index("s")
        b = sid * 2 + cid

        pltpu.sync_copy(x_hbm.at[b, :, :], x_v.at[:, 0:C])
        pltpu.sync_copy(idx_hbm, idx_v)

        zero16 = jnp.zeros((_L,), jnp.float32)

        def zero_body(i, carry):
            w_v[pl.ds(i * _L, _L)] = zero16
            return carry

        lax.fori_loop(0, T // _L, zero_body, 0)

        lane = lax.iota(jnp.int32, _L)

        def group_body(g, carry):
            t0 = g * _L
            tvec = t0 + lane
            tk = tvec * K
            idxv = [plsc.load_gather(idx_v, [tk + j]) for j in range(K)]

            def c_body(c, acc):
                q2 = acc[0]
                dots = list(acc[1:1 + K])
                n2s = list(acc[1 + K:])
                cvec = jnp.full((_L,), c, jnp.int32)
                qv = plsc.load_gather(x_v, [tvec, cvec])
                q2 = q2 + qv * qv
                for j in range(K):
                    nv = plsc.load_gather(x_v, [idxv[j], cvec])
                    dots[j] = dots[j] + qv * nv
                    n2s[j] = n2s[j] + nv * nv
                return tuple([q2] + dots + n2s)

            init = tuple(zero16 for _ in range(1 + 2 * K))
            acc = lax.fori_loop(0, C, c_body, init, unroll=4)
            q2 = acc[0]
            dots = acc[1:1 + K]
            n2s = acc[1 + K:]

            sims = [
                dots[j] * _fast_rsqrt(jnp.maximum(q2 * n2s[j], 1e-16))
                for j in range(K)
            ]
            m = sims[0]
            for j in range(1, K):
                m = jnp.maximum(m, sims[j])
            exps = [jnp.exp(s - m) for s in sims]
            tot = exps[0]
            for j in range(1, K):
                tot = tot + exps[j]
            r = 1.0 / tot
            for j in range(K):
                plsc.addupdate_scatter(w_v, [idxv[j]], exps[j] * r)
            return carry

        lax.fori_loop(0, n_groups, group_body, 0)

        def row_body(s, acc):
            wv = plsc.load_gather(w_v, [jnp.full((_L,), s, jnp.int32)])
            return tuple(
                acc[k] + wv * x_v[s, pl.ds(k * _L, _L)]
                for k in range(c_chunks)
            )

        acc = lax.fori_loop(
            0, T, row_body, tuple(zero16 for _ in range(c_chunks)), unroll=4
        )
        scale = jnp.float32(1.0 / T)
        for k in range(c_chunks):
            o_v[pl.ds(k * _L, _L)] = acc[k] * scale
        pltpu.sync_copy(o_v, out_hbm.at[b])

    return sc_kernel


def kernel(X, neighbor_idx):
    B, T, C = X.shape
    K = neighbor_idx.shape[1]
    idx_flat = neighbor_idx.astype(jnp.int32).reshape(T * K)
    out = _make_sc_kernel(B, T, C, K)(X, idx_flat)
    return out.reshape(B, 1, C)


# speedup vs baseline: 2.5601x; 1.0995x over previous
import functools

import jax
import jax.numpy as jnp
from jax import lax
from jax.experimental import pallas as pl
from jax.experimental.pallas import tpu as pltpu
from jax.experimental.pallas import tpu_sc as plsc

_L = 16


def _fast_rsqrt(x):
    i = lax.bitcast_convert_type(x, jnp.int32)
    i = jnp.int32(0x5F3759DF) - lax.shift_right_arithmetic(i, 1)
    y = lax.bitcast_convert_type(i, jnp.float32)
    for _ in range(3):
        y = y * (1.5 - 0.5 * x * y * y)
    return y


def _make_sc_kernel(B, T, C, K):
    assert C % _L == 0
    n_groups = T // _L
    c_chunks = C // _L
    mesh = plsc.VectorSubcoreMesh(
        core_axis_name="c", subcore_axis_name="s", num_cores=2, num_subcores=16
    )

    @functools.partial(
        pl.kernel,
        out_type=jax.ShapeDtypeStruct((B, C), jnp.float32),
        mesh=mesh,
        scratch_types=[
            pltpu.VMEM((T, C + 1), jnp.float32),
            pltpu.VMEM((T, K), jnp.int32),
            pltpu.VMEM((T,), jnp.float32),
            pltpu.VMEM((T,), jnp.float32),
            pltpu.VMEM((C,), jnp.float32),
        ],
        compiler_params=pltpu.CompilerParams(
            needs_layout_passes=False, use_tc_tiling_on_sc=False
        ),
    )
    def sc_kernel(x_hbm, idx_hbm, out_hbm, x_v, idx_v, n2_v, w_v, o_v):
        cid = lax.axis_index("c")
        sid = lax.axis_index("s")
        b = sid * 2 + cid

        pltpu.sync_copy(x_hbm.at[b, :, :], x_v.at[:, 0:C])
        pltpu.sync_copy(idx_hbm, idx_v)

        zero16 = jnp.zeros((_L,), jnp.float32)
        lane = lax.iota(jnp.int32, _L)

        def norm_body(g, carry):
            rvec = g * _L + lane

            def nc_body(c, n2):
                v = plsc.load_gather(x_v, [rvec, jnp.full((_L,), c, jnp.int32)])
                return n2 + v * v

            n2 = lax.fori_loop(0, C, nc_body, zero16, unroll=4)
            n2_v[pl.ds(g * _L, _L)] = n2
            w_v[pl.ds(g * _L, _L)] = zero16
            return carry

        lax.fori_loop(0, n_groups, norm_body, 0)

        def group_body(g, carry):
            t0 = g * _L
            tvec = t0 + lane
            idxv = [
                plsc.load_gather(idx_v, [tvec, jnp.full((_L,), j, jnp.int32)])
                for j in range(K)
            ]

            def c_body(c, dots):
                cvec = jnp.full((_L,), c, jnp.int32)
                qv = plsc.load_gather(x_v, [tvec, cvec])
                return tuple(
                    dots[j] + qv * plsc.load_gather(x_v, [idxv[j], cvec])
                    for j in range(K)
                )

            dots = lax.fori_loop(
                0, C, c_body, tuple(zero16 for _ in range(K)), unroll=4
            )

            q2 = n2_v[pl.ds(t0, _L)]
            n2s = [plsc.load_gather(n2_v, [idxv[j]]) for j in range(K)]
            sims = [
                dots[j] * _fast_rsqrt(jnp.maximum(q2 * n2s[j], 1e-16))
                for j in range(K)
            ]
            m = sims[0]
            for j in range(1, K):
                m = jnp.maximum(m, sims[j])
            exps = [jnp.exp(s - m) for s in sims]
            tot = exps[0]
            for j in range(1, K):
                tot = tot + exps[j]
            r = 1.0 / tot
            for j in range(K):
                plsc.addupdate_scatter(w_v, [idxv[j]], exps[j] * r)
            return carry

        lax.fori_loop(0, n_groups, group_body, 0)

        def row_body(s, acc):
            wv = plsc.load_gather(w_v, [jnp.full((_L,), s, jnp.int32)])
            return tuple(
                acc[k] + wv * x_v[s, pl.ds(k * _L, _L)]
                for k in range(c_chunks)
            )

        acc = lax.fori_loop(
            0, T, row_body, tuple(zero16 for _ in range(c_chunks)), unroll=4
        )
        scale = jnp.float32(1.0 / T)
        for k in range(c_chunks):
            o_v[pl.ds(k * _L, _L)] = acc[k] * scale
        pltpu.sync_copy(o_v, out_hbm.at[b])

    return sc_kernel


def kernel(X, neighbor_idx):
    B, T, C = X.shape
    K = neighbor_idx.shape[1]
    idx32 = neighbor_idx.astype(jnp.int32)
    out = _make_sc_kernel(B, T, C, K)(X, idx32)
    return out.reshape(B, 1, C)
